# baseline (speedup 1.0000x reference)
import functools

import jax
import jax.numpy as jnp
from jax import lax
from jax.experimental import pallas as pl
from jax.experimental.pallas import tpu as pltpu
from jax.experimental.pallas import tpu_sc as plsc

N_NODES = 10000
N_EDGES = 320000
D = 128

NC = 2
NS = 16
NW = NC * NS
E_PER_W = N_EDGES // NW
CHUNK = 40
N_CHUNKS = E_PER_W // CHUNK
NBUF = 5
ROWS_PER_S = N_NODES // NS

@functools.cache
def _build_sc_kernels():
    mesh = plsc.VectorSubcoreMesh(core_axis_name="c", subcore_axis_name="s")

    deg_kernel = functools.partial(
        pl.kernel,
        mesh=mesh,
        compiler_params=pltpu.CompilerParams(needs_layout_passes=False),
        out_type=[
            jax.ShapeDtypeStruct((NC, NS, N_NODES), jnp.float32),
            jax.ShapeDtypeStruct((NC, NS, N_NODES), jnp.float32),
        ],
        scratch_types=[
            pltpu.VMEM((E_PER_W,), jnp.int32),
            pltpu.VMEM((E_PER_W,), jnp.int32),
            pltpu.VMEM((N_NODES,), jnp.float32),
            pltpu.VMEM((N_NODES,), jnp.float32),
        ],
    )(_deg_body)

    segsum_kernel = functools.partial(
        pl.kernel,
        mesh=mesh,
        out_type=jax.ShapeDtypeStruct((NC, NS, ROWS_PER_S, D), jnp.float32),
        scratch_types=[
            pltpu.VMEM((NBUF, 2, CHUNK), jnp.int32),
            pltpu.VMEM((NBUF, CHUNK, D), jnp.float32),
            pltpu.VMEM_SHARED((N_NODES, D), jnp.float32),
        ] + [pltpu.SemaphoreType.DMA] * (3 * NBUF),
    )(_segsum_body)

    return deg_kernel, segsum_kernel


def _deg_body(src_hbm, dst_hbm, zer_hbm, dego_hbm, degi_hbm,
              sidx_v, didx_v, ho_v, hi_v):
    c = lax.axis_index("c")
    s = lax.axis_index("s")
    wid = s * NC + c
    base = wid * E_PER_W

    pltpu.sync_copy(src_hbm.at[pl.ds(base, E_PER_W)], sidx_v)
    pltpu.sync_copy(dst_hbm.at[pl.ds(base, E_PER_W)], didx_v)
    pltpu.sync_copy(zer_hbm, ho_v)
    pltpu.sync_copy(zer_hbm, hi_v)

    one16 = jnp.ones((16,), jnp.float32)

    def body(g, carry):
        off = g * 16
        plsc.addupdate_scatter(ho_v, [sidx_v[pl.ds(off, 16)]], one16)
        plsc.addupdate_scatter(hi_v, [didx_v[pl.ds(off, 16)]], one16)
        return carry

    lax.fori_loop(0, E_PER_W // 16, body, 0)

    pltpu.sync_copy(ho_v, dego_hbm.at[c, s])
    pltpu.sync_copy(hi_v, degi_hbm.at[c, s])


def _segsum_body(h_hbm, srcg_hbm, dstg_hbm, zer_hbm, out_hbm,
                 idx_v, rows_v, agg_sh, *sems):
    c = lax.axis_index("c")
    s = lax.axis_index("s")
    wid = s * NC + c

    for j in range(5):
        pltpu.sync_copy(
            zer_hbm, agg_sh.at[pl.ds(s * ROWS_PER_S + j * 125, 125)])
    plsc.subcore_barrier()

    def body(k, carry):
        ips, dps, cps = [], [], []
        for u in range(NBUF):
            g = k * NBUF + u
            ips.append(pltpu.async_copy(
                srcg_hbm.at[wid, g], idx_v.at[u, 0], sems[NBUF + u]))
            dps.append(pltpu.async_copy(
                dstg_hbm.at[wid, g], idx_v.at[u, 1], sems[2 * NBUF + u]))
        for u in range(NBUF):
            ips[u].wait()
            cps.append(pltpu.async_copy(
                h_hbm.at[idx_v.at[u, 0]], rows_v.at[u], sems[u]))
        for u in range(NBUF):
            dps[u].wait()
            cps[u].wait()
            pltpu.sync_copy(rows_v.at[u], agg_sh.at[idx_v.at[u, 1]], add=True)
        return carry

    lax.fori_loop(0, N_CHUNKS // NBUF, body, 0)
    plsc.subcore_barrier()

    sl = pl.ds(s * ROWS_PER_S, ROWS_PER_S)
    pltpu.sync_copy(agg_sh.at[sl], out_hbm.at[c, s])


def _mm1_body(x_ref, w1_ref, hoT_ref, hiT_ref, o_ref, no_ref, ni_ref):
    no = lax.rsqrt(jnp.maximum(jnp.sum(hoT_ref[...], axis=1), 1.0))[:, None]
    ni = lax.rsqrt(jnp.maximum(jnp.sum(hiT_ref[...], axis=1), 1.0))[:, None]
    no_ref[...] = no
    ni_ref[...] = ni
    o_ref[...] = jnp.dot(x_ref[...] * no, w1_ref[...],
                         preferred_element_type=jnp.float32)


def _mm2_body(a_ref, no_ref, ni_ref, b1_ref, w2_ref, o_ref):
    a = a_ref[0] + a_ref[1]
    h1 = jnp.maximum(a * ni_ref[...] + b1_ref[...], 0.0)
    o_ref[...] = jnp.dot(h1 * no_ref[...], w2_ref[...],
                         preferred_element_type=jnp.float32)


def _final_body(a_ref, ni_ref, b2_ref, o_ref):
    a = a_ref[0] + a_ref[1]
    o_ref[...] = a * ni_ref[...] + b2_ref[...]


_BLK = 1000
_GRID = N_NODES // _BLK

_spec_nd = pl.BlockSpec((_BLK, D), lambda i: (i, 0))
_spec_w = pl.BlockSpec((D, D), lambda i: (0, 0))
_spec_b = pl.BlockSpec((1, D), lambda i: (0, 0))
_spec_agg = pl.BlockSpec((NC, _BLK, D), lambda i: (0, i, 0))
_spec_norm = pl.BlockSpec((_BLK, 1), lambda i: (i, 0))
_out_nd = jax.ShapeDtypeStruct((N_NODES, D), jnp.float32)


def kernel(x, edge_index, W1, b1, W2, b2):
    zer128 = jnp.zeros((125, D), jnp.float32)
    zer1d = jnp.zeros((N_NODES,), jnp.float32)
    edge_index = edge_index.astype(jnp.int32)
    src = edge_index[0]
    dst = edge_index[1]
    deg_kernel, segsum_kernel = _build_sc_kernels()

    srcg = src.reshape(NW, N_CHUNKS, CHUNK)
    dstg = dst.reshape(NW, N_CHUNKS, CHUNK)

    dego, degi = deg_kernel(src, dst, zer1d)
    degoT = dego.reshape(NW, N_NODES).T
    degiT = degi.reshape(NW, N_NODES).T

    _spec_degT = pl.BlockSpec((_BLK, NW), lambda i: (i, 0))
    h0, no, ni = pl.pallas_call(
        _mm1_body,
        grid=(_GRID,),
        in_specs=[_spec_nd, _spec_w, _spec_degT, _spec_degT],
        out_specs=[_spec_nd, _spec_norm, _spec_norm],
        out_shape=[_out_nd,
                   jax.ShapeDtypeStruct((N_NODES, 1), jnp.float32),
                   jax.ShapeDtypeStruct((N_NODES, 1), jnp.float32)],
    )(x, W1, degoT, degiT)

    agg1 = segsum_kernel(h0, srcg, dstg, zer128).reshape(NC, N_NODES, D)

    h2 = pl.pallas_call(
        _mm2_body,
        grid=(_GRID,),
        in_specs=[_spec_agg, _spec_norm, _spec_norm, _spec_b, _spec_w],
        out_specs=_spec_nd,
        out_shape=_out_nd,
    )(agg1, no, ni, b1.reshape(1, D), W2)

    agg2 = segsum_kernel(h2, srcg, dstg, zer128).reshape(NC, N_NODES, D)

    out = pl.pallas_call(
        _final_body,
        grid=(_GRID,),
        in_specs=[_spec_agg, _spec_norm, _spec_b],
        out_specs=_spec_nd,
        out_shape=_out_nd,
    )(agg2, ni, b2.reshape(1, D))

    return out


# speedup vs baseline: 1.0004x; 1.0004x over previous
import functools

import jax
import jax.numpy as jnp
from jax import lax
from jax.experimental import pallas as pl
from jax.experimental.pallas import tpu as pltpu
from jax.experimental.pallas import tpu_sc as plsc

N_NODES = 10000
N_EDGES = 320000
D = 128

NC = 2
NS = 16
NW = NC * NS
E_PER_W = N_EDGES // NW
CHUNK = 40
N_CHUNKS = E_PER_W // CHUNK
NBUF = 5
N_MAIN = (N_CHUNKS // NBUF) * NBUF
N_TAIL = N_CHUNKS - N_MAIN
ROWS_PER_S = N_NODES // NS

@functools.cache
def _build_sc_kernels():
    mesh = plsc.VectorSubcoreMesh(core_axis_name="c", subcore_axis_name="s")

    deg_kernel = functools.partial(
        pl.kernel,
        mesh=mesh,
        compiler_params=pltpu.CompilerParams(needs_layout_passes=False),
        out_type=[
            jax.ShapeDtypeStruct((NC, NS, N_NODES), jnp.float32),
            jax.ShapeDtypeStruct((NC, NS, N_NODES), jnp.float32),
        ],
        scratch_types=[
            pltpu.VMEM((E_PER_W,), jnp.int32),
            pltpu.VMEM((E_PER_W,), jnp.int32),
            pltpu.VMEM((N_NODES,), jnp.float32),
            pltpu.VMEM((N_NODES,), jnp.float32),
        ],
    )(_deg_body)

    segsum_kernel = functools.partial(
        pl.kernel,
        mesh=mesh,
        out_type=jax.ShapeDtypeStruct((NC, NS, ROWS_PER_S, D), jnp.float32),
        scratch_types=[
            pltpu.VMEM((NBUF, 2, CHUNK), jnp.int32),
            pltpu.VMEM((NBUF, CHUNK, D), jnp.float32),
            pltpu.VMEM_SHARED((N_NODES, D), jnp.float32),
        ] + [pltpu.SemaphoreType.DMA] * (3 * NBUF),
    )(_segsum_body)

    return deg_kernel, segsum_kernel


def _deg_body(src_hbm, dst_hbm, zer_hbm, dego_hbm, degi_hbm,
              sidx_v, didx_v, ho_v, hi_v):
    c = lax.axis_index("c")
    s = lax.axis_index("s")
    wid = s * NC + c
    base = wid * E_PER_W

    pltpu.sync_copy(src_hbm.at[pl.ds(base, E_PER_W)], sidx_v)
    pltpu.sync_copy(dst_hbm.at[pl.ds(base, E_PER_W)], didx_v)
    pltpu.sync_copy(zer_hbm, ho_v)
    pltpu.sync_copy(zer_hbm, hi_v)

    one16 = jnp.ones((16,), jnp.float32)

    def body(g, carry):
        off = g * 16
        plsc.addupdate_scatter(ho_v, [sidx_v[pl.ds(off, 16)]], one16)
        plsc.addupdate_scatter(hi_v, [didx_v[pl.ds(off, 16)]], one16)
        return carry

    lax.fori_loop(0, E_PER_W // 16, body, 0)

    pltpu.sync_copy(ho_v, dego_hbm.at[c, s])
    pltpu.sync_copy(hi_v, degi_hbm.at[c, s])


def _segsum_body(h_hbm, srcg_hbm, dstg_hbm, zer_hbm, out_hbm,
                 idx_v, rows_v, agg_sh, *sems):
    c = lax.axis_index("c")
    s = lax.axis_index("s")
    wid = s * NC + c

    for j in range(5):
        pltpu.sync_copy(
            zer_hbm, agg_sh.at[pl.ds(s * ROWS_PER_S + j * 125, 125)])
    plsc.subcore_barrier()

    def body(k, carry):
        ips, dps, cps = [], [], []
        for u in range(NBUF):
            g = k * NBUF + u
            ips.append(pltpu.async_copy(
                srcg_hbm.at[wid, g], idx_v.at[u, 0], sems[NBUF + u]))
            dps.append(pltpu.async_copy(
                dstg_hbm.at[wid, g], idx_v.at[u, 1], sems[2 * NBUF + u]))
        for u in range(NBUF):
            ips[u].wait()
            cps.append(pltpu.async_copy(
                h_hbm.at[idx_v.at[u, 0]], rows_v.at[u], sems[u]))
        for u in range(NBUF):
            dps[u].wait()
            cps[u].wait()
            pltpu.sync_copy(rows_v.at[u], agg_sh.at[idx_v.at[u, 1]], add=True)
        return carry

    lax.fori_loop(0, N_MAIN // NBUF, body, 0)
    for t in range(N_TAIL):
        g = N_MAIN + t
        pltpu.sync_copy(srcg_hbm.at[wid, g], idx_v.at[0, 0])
        pltpu.sync_copy(dstg_hbm.at[wid, g], idx_v.at[0, 1])
        pltpu.async_copy(
            h_hbm.at[idx_v.at[0, 0]], rows_v.at[0], sems[0]).wait()
        pltpu.sync_copy(rows_v.at[0], agg_sh.at[idx_v.at[0, 1]], add=True)
    plsc.subcore_barrier()

    sl = pl.ds(s * ROWS_PER_S, ROWS_PER_S)
    pltpu.sync_copy(agg_sh.at[sl], out_hbm.at[c, s])


def _mm1_body(x_ref, w1_ref, hoT_ref, hiT_ref, o_ref, no_ref, ni_ref):
    no = lax.rsqrt(jnp.maximum(jnp.sum(hoT_ref[...], axis=1), 1.0))[:, None]
    ni = lax.rsqrt(jnp.maximum(jnp.sum(hiT_ref[...], axis=1), 1.0))[:, None]
    no_ref[...] = no
    ni_ref[...] = ni
    o_ref[...] = jnp.dot(x_ref[...] * no, w1_ref[...],
                         preferred_element_type=jnp.float32)


def _mm2_body(a_ref, no_ref, ni_ref, b1_ref, w2_ref, o_ref):
    a = a_ref[0] + a_ref[1]
    h1 = jnp.maximum(a * ni_ref[...] + b1_ref[...], 0.0)
    o_ref[...] = jnp.dot(h1 * no_ref[...], w2_ref[...],
                         preferred_element_type=jnp.float32)


def _final_body(a_ref, ni_ref, b2_ref, o_ref):
    a = a_ref[0] + a_ref[1]
    o_ref[...] = a * ni_ref[...] + b2_ref[...]


_BLK = 1000
_GRID = N_NODES // _BLK

_spec_nd = pl.BlockSpec((_BLK, D), lambda i: (i, 0))
_spec_w = pl.BlockSpec((D, D), lambda i: (0, 0))
_spec_b = pl.BlockSpec((1, D), lambda i: (0, 0))
_spec_agg = pl.BlockSpec((NC, _BLK, D), lambda i: (0, i, 0))
_spec_norm = pl.BlockSpec((_BLK, 1), lambda i: (i, 0))
_out_nd = jax.ShapeDtypeStruct((N_NODES, D), jnp.float32)


def kernel(x, edge_index, W1, b1, W2, b2):
    zer128 = jnp.zeros((125, D), jnp.float32)
    zer1d = jnp.zeros((N_NODES,), jnp.float32)
    edge_index = edge_index.astype(jnp.int32)
    src = edge_index[0]
    dst = edge_index[1]
    deg_kernel, segsum_kernel = _build_sc_kernels()

    srcg = src.reshape(NW, N_CHUNKS, CHUNK)
    dstg = dst.reshape(NW, N_CHUNKS, CHUNK)

    dego, degi = deg_kernel(src, dst, zer1d)
    degoT = dego.reshape(NW, N_NODES).T
    degiT = degi.reshape(NW, N_NODES).T

    _spec_degT = pl.BlockSpec((_BLK, NW), lambda i: (i, 0))
    h0, no, ni = pl.pallas_call(
        _mm1_body,
        grid=(_GRID,),
        in_specs=[_spec_nd, _spec_w, _spec_degT, _spec_degT],
        out_specs=[_spec_nd, _spec_norm, _spec_norm],
        out_shape=[_out_nd,
                   jax.ShapeDtypeStruct((N_NODES, 1), jnp.float32),
                   jax.ShapeDtypeStruct((N_NODES, 1), jnp.float32)],
    )(x, W1, degoT, degiT)

    agg1 = segsum_kernel(h0, srcg, dstg, zer128).reshape(NC, N_NODES, D)

    h2 = pl.pallas_call(
        _mm2_body,
        grid=(_GRID,),
        in_specs=[_spec_agg, _spec_norm, _spec_norm, _spec_b, _spec_w],
        out_specs=_spec_nd,
        out_shape=_out_nd,
    )(agg1, no, ni, b1.reshape(1, D), W2)

    agg2 = segsum_kernel(h2, srcg, dstg, zer128).reshape(NC, N_NODES, D)

    out = pl.pallas_call(
        _final_body,
        grid=(_GRID,),
        in_specs=[_spec_agg, _spec_norm, _spec_b],
        out_specs=_spec_nd,
        out_shape=_out_nd,
    )(agg2, ni, b2.reshape(1, D))

    return out


# speedup vs baseline: 1.0248x; 1.0245x over previous
import functools

import jax
import jax.numpy as jnp
from jax import lax
from jax.experimental import pallas as pl
from jax.experimental.pallas import tpu as pltpu
from jax.experimental.pallas import tpu_sc as plsc

N_NODES = 10000
N_EDGES = 320000
D = 128

NC = 2
NS = 16
NW = NC * NS
E_PER_W = N_EDGES // NW
CHUNK = 40
N_CHUNKS = E_PER_W // CHUNK
NBUF = 5
N_MAIN = (N_CHUNKS // NBUF) * NBUF
N_TAIL = N_CHUNKS - N_MAIN
ROWS_PER_S = N_NODES // NS

@functools.cache
def _build_sc_kernels():
    mesh = plsc.VectorSubcoreMesh(core_axis_name="c", subcore_axis_name="s")

    deg_kernel = functools.partial(
        pl.kernel,
        mesh=mesh,
        compiler_params=pltpu.CompilerParams(needs_layout_passes=False),
        out_type=[
            jax.ShapeDtypeStruct((NC, NS, N_NODES), jnp.float32),
            jax.ShapeDtypeStruct((NC, NS, N_NODES), jnp.float32),
        ],
        scratch_types=[
            pltpu.VMEM((E_PER_W,), jnp.int32),
            pltpu.VMEM((E_PER_W,), jnp.int32),
            pltpu.VMEM((N_NODES,), jnp.float32),
            pltpu.VMEM((N_NODES,), jnp.float32),
        ],
    )(_deg_body)

    segsum_kernel = functools.partial(
        pl.kernel,
        mesh=mesh,
        out_type=jax.ShapeDtypeStruct((NC, NS, ROWS_PER_S, D), jnp.float32),
        scratch_types=[
            pltpu.VMEM((NBUF, 2, CHUNK), jnp.int32),
            pltpu.VMEM((NBUF, CHUNK, D), jnp.float32),
            pltpu.VMEM_SHARED((N_NODES, D), jnp.float32),
        ] + [pltpu.SemaphoreType.DMA] * (3 * NBUF),
    )(_segsum_body)

    return deg_kernel, segsum_kernel


def _deg_body(src_hbm, dst_hbm, zer_hbm, dego_hbm, degi_hbm,
              sidx_v, didx_v, ho_v, hi_v):
    c = lax.axis_index("c")
    s = lax.axis_index("s")
    wid = s * NC + c
    base = wid * E_PER_W

    pltpu.sync_copy(src_hbm.at[pl.ds(base, E_PER_W)], sidx_v)
    pltpu.sync_copy(dst_hbm.at[pl.ds(base, E_PER_W)], didx_v)
    pltpu.sync_copy(zer_hbm, ho_v)
    pltpu.sync_copy(zer_hbm, hi_v)

    one16 = jnp.ones((16,), jnp.float32)

    def body(g, carry):
        off = g * 16
        plsc.addupdate_scatter(ho_v, [sidx_v[pl.ds(off, 16)]], one16)
        plsc.addupdate_scatter(hi_v, [didx_v[pl.ds(off, 16)]], one16)
        return carry

    lax.fori_loop(0, E_PER_W // 16, body, 0)

    pltpu.sync_copy(ho_v, dego_hbm.at[c, s])
    pltpu.sync_copy(hi_v, degi_hbm.at[c, s])


def _segsum_body(h_hbm, srcg_hbm, dstg_hbm, zer_hbm, out_hbm,
                 idx_v, rows_v, agg_sh, *sems):
    c = lax.axis_index("c")
    s = lax.axis_index("s")
    wid = s * NC + c

    for j in range(5):
        pltpu.sync_copy(
            zer_hbm, agg_sh.at[pl.ds(s * ROWS_PER_S + j * 125, 125)])
    plsc.subcore_barrier()

    def body(k, carry):
        ips, dps, cps = [], [], []
        for u in range(NBUF):
            g = k * NBUF + u
            ips.append(pltpu.async_copy(
                srcg_hbm.at[wid, g], idx_v.at[u, 0], sems[NBUF + u]))
            dps.append(pltpu.async_copy(
                dstg_hbm.at[wid, g], idx_v.at[u, 1], sems[2 * NBUF + u]))
        for u in range(NBUF):
            ips[u].wait()
            cps.append(pltpu.async_copy(
                h_hbm.at[idx_v.at[u, 0]], rows_v.at[u], sems[u]))
        for u in range(NBUF):
            dps[u].wait()
            cps[u].wait()
            pltpu.sync_copy(rows_v.at[u], agg_sh.at[idx_v.at[u, 1]], add=True)
        return carry

    lax.fori_loop(0, N_MAIN // NBUF, body, 0)
    for t in range(N_TAIL):
        g = N_MAIN + t
        pltpu.sync_copy(srcg_hbm.at[wid, g], idx_v.at[0, 0])
        pltpu.sync_copy(dstg_hbm.at[wid, g], idx_v.at[0, 1])
        pltpu.async_copy(
            h_hbm.at[idx_v.at[0, 0]], rows_v.at[0], sems[0]).wait()
        pltpu.sync_copy(rows_v.at[0], agg_sh.at[idx_v.at[0, 1]], add=True)
    plsc.subcore_barrier()

    sl = pl.ds(s * ROWS_PER_S, ROWS_PER_S)
    pltpu.sync_copy(agg_sh.at[sl], out_hbm.at[c, s])


def _norms_body(ho_ref, hi_ref, no_ref, ni_ref):
    dego = jnp.sum(ho_ref[...], axis=(0, 1))[:, None]
    degi = jnp.sum(hi_ref[...], axis=(0, 1))[:, None]
    no_ref[...] = lax.rsqrt(jnp.maximum(dego, 1.0))
    ni_ref[...] = lax.rsqrt(jnp.maximum(degi, 1.0))


def _mm1_body(x_ref, w1_ref, no_ref, o_ref):
    o_ref[...] = jnp.dot(x_ref[...] * no_ref[...], w1_ref[...],
                         preferred_element_type=jnp.float32)


def _mm2_body(a_ref, no_ref, ni_ref, b1_ref, w2_ref, o_ref):
    a = a_ref[0] + a_ref[1]
    h1 = jnp.maximum(a * ni_ref[...] + b1_ref[...], 0.0)
    o_ref[...] = jnp.dot(h1 * no_ref[...], w2_ref[...],
                         preferred_element_type=jnp.float32)


def _final_body(a_ref, ni_ref, b2_ref, o_ref):
    a = a_ref[0] + a_ref[1]
    o_ref[...] = a * ni_ref[...] + b2_ref[...]


_BLK = 1000
_GRID = N_NODES // _BLK

_spec_nd = pl.BlockSpec((_BLK, D), lambda i: (i, 0))
_spec_w = pl.BlockSpec((D, D), lambda i: (0, 0))
_spec_b = pl.BlockSpec((1, D), lambda i: (0, 0))
_spec_agg = pl.BlockSpec((NC, _BLK, D), lambda i: (0, i, 0))
_spec_norm = pl.BlockSpec((_BLK, 1), lambda i: (i, 0))
_out_nd = jax.ShapeDtypeStruct((N_NODES, D), jnp.float32)


def kernel(x, edge_index, W1, b1, W2, b2):
    zer128 = jnp.zeros((125, D), jnp.float32)
    zer1d = jnp.zeros((N_NODES,), jnp.float32)
    edge_index = edge_index.astype(jnp.int32)
    src = edge_index[0]
    dst = edge_index[1]
    deg_kernel, segsum_kernel = _build_sc_kernels()

    srcg = src.reshape(NW, N_CHUNKS, CHUNK)
    dstg = dst.reshape(NW, N_CHUNKS, CHUNK)

    dego, degi = deg_kernel(src, dst, zer1d)

    no, ni = pl.pallas_call(
        _norms_body,
        grid=(1,),
        in_specs=[pl.BlockSpec((NC, NS, N_NODES), lambda i: (0, 0, 0))] * 2,
        out_specs=[pl.BlockSpec((N_NODES, 1), lambda i: (0, 0))] * 2,
        out_shape=[jax.ShapeDtypeStruct((N_NODES, 1), jnp.float32)] * 2,
    )(dego, degi)

    h0 = pl.pallas_call(
        _mm1_body,
        grid=(_GRID,),
        in_specs=[_spec_nd, _spec_w, _spec_norm],
        out_specs=_spec_nd,
        out_shape=_out_nd,
    )(x, W1, no)

    agg1 = segsum_kernel(h0, srcg, dstg, zer128).reshape(NC, N_NODES, D)

    h2 = pl.pallas_call(
        _mm2_body,
        grid=(_GRID,),
        in_specs=[_spec_agg, _spec_norm, _spec_norm, _spec_b, _spec_w],
        out_specs=_spec_nd,
        out_shape=_out_nd,
    )(agg1, no, ni, b1.reshape(1, D), W2)

    agg2 = segsum_kernel(h2, srcg, dstg, zer128).reshape(NC, N_NODES, D)

    out = pl.pallas_call(
        _final_body,
        grid=(_GRID,),
        in_specs=[_spec_agg, _spec_norm, _spec_b],
        out_specs=_spec_nd,
        out_shape=_out_nd,
    )(agg2, ni, b2.reshape(1, D))

    return out
